# baseline (speedup 1.0000x reference)
import jax
import jax.numpy as jnp
from jax import lax
from jax.experimental import pallas as pl
from jax.experimental.pallas import tpu as pltpu
from jax.experimental.pallas import tpu_sc as plsc

EMB = 64
SCALE = 8.0

NUM_CORES = 2
NUM_SUBCORES = 16
NUM_WORKERS = NUM_CORES * NUM_SUBCORES

BATCH = 4096
SEQ = 200
ROWS_PER_W = BATCH // NUM_WORKERS
R_STEP = 2
STEPS = ROWS_PER_W // R_STEP
NBUF = 3
SPLIT = (0, 104, SEQ)


def _body(table_hbm, tok_hbm, out_hbm,
          idx_all, rows0, rows1, rows2,
          gsem0, gsem1, gsem2, osem0, osem1, osem2):
    rows = (rows0, rows1, rows2)
    gsem = (gsem0, gsem1, gsem2)
    osem = (osem0, osem1, osem2)

    wid = lax.axis_index("s") * NUM_CORES + lax.axis_index("c")
    base_brow = wid * ROWS_PER_W

    pltpu.sync_copy(tok_hbm.at[pl.ds(base_brow, ROWS_PER_W)], idx_all)

    def fire_gathers(h, d):
        for rr in range(R_STEP):
            for j in range(len(SPLIT) - 1):
                off, w = SPLIT[j], SPLIT[j + 1] - SPLIT[j]
                pltpu.async_copy(
                    table_hbm.at[idx_all.at[h * R_STEP + rr, pl.ds(off, w)]],
                    rows[d].at[rr, pl.ds(off, w)],
                    gsem[d],
                )

    def wait_gathers(d):
        pltpu.make_async_copy(
            out_hbm.at[pl.ds(0, R_STEP)], rows[d], gsem[d]).wait()

    def fire_writeout(h, d):
        row0 = base_brow + h * R_STEP
        pltpu.async_copy(rows[d], out_hbm.at[pl.ds(row0, R_STEP)], osem[d])

    def wait_writeout(d):
        pltpu.make_async_copy(
            rows[d], out_hbm.at[pl.ds(0, R_STEP)], osem[d]).wait()

    def scale(d):
        def sbody(r, c):
            for tt in range(4):
                for rr in range(R_STEP):
                    for cc in range(EMB // 16):
                        sl = (rr, r * 4 + tt, pl.ds(cc * 16, 16))
                        rows[d][sl] = rows[d][sl] * SCALE
            return c

        lax.fori_loop(0, SEQ // 4, sbody, 0, unroll=False)

    def pipe_step(h, d, first_round):
        wait_gathers(d)
        d2 = (d + 1) % NBUF
        if not first_round:
            wait_writeout(d2)
        fire_gathers(h + 1, d2)
        scale(d)
        fire_writeout(h, d)

    fire_gathers(0, 0)
    pipe_step(0, 0, True)
    pipe_step(1, 1, True)
    pipe_step(2, 2, False)

    def loop_body(i, c):
        h = i * NBUF
        pipe_step(h, 0, False)
        pipe_step(h + 1, 1, False)
        pipe_step(h + 2, 2, False)
        return c

    lax.fori_loop(1, 1 + (STEPS - 1 - NBUF) // NBUF, loop_body, 0,
                  unroll=False)

    h = STEPS - 1
    wait_gathers(0)
    scale(0)
    fire_writeout(h, 0)

    wait_writeout(1)
    wait_writeout(2)
    wait_writeout(0)


def kernel(tokens, table):
    mesh = plsc.VectorSubcoreMesh(
        core_axis_name="c", subcore_axis_name="s",
        num_cores=NUM_CORES, num_subcores=NUM_SUBCORES,
    )
    out = pl.kernel(
        _body,
        out_type=jax.ShapeDtypeStruct((BATCH, SEQ, EMB), jnp.float32),
        mesh=mesh,
        compiler_params=pltpu.CompilerParams(use_tc_tiling_on_sc=False),
        scratch_types=[
            pltpu.VMEM((ROWS_PER_W, SEQ), jnp.int32),
            pltpu.VMEM((R_STEP, SEQ, EMB), jnp.float32),
            pltpu.VMEM((R_STEP, SEQ, EMB), jnp.float32),
            pltpu.VMEM((R_STEP, SEQ, EMB), jnp.float32),
            pltpu.SemaphoreType.DMA,
            pltpu.SemaphoreType.DMA,
            pltpu.SemaphoreType.DMA,
            pltpu.SemaphoreType.DMA,
            pltpu.SemaphoreType.DMA,
            pltpu.SemaphoreType.DMA,
        ],
    )(table, tokens.astype(jnp.int32))
    return out


# speedup vs baseline: 1.1615x; 1.1615x over previous
import jax
import jax.numpy as jnp
from jax import lax
from jax.experimental import pallas as pl
from jax.experimental.pallas import tpu as pltpu
from jax.experimental.pallas import tpu_sc as plsc

EMB = 64
PADW = 128
SCALE = 8.0

NUM_CORES = 2
NUM_SUBCORES = 16
NUM_WORKERS = NUM_CORES * NUM_SUBCORES

N_TOK = 4096 * 200
TOK_PER_W = N_TOK // NUM_WORKERS
STEP = 128
STEPS = TOK_PER_W // STEP
IDX_ROWS = TOK_PER_W // STEP


def _body(table_hbm, tok_hbm, out_hbm,
          idx_all, gbuf0, gbuf1, obuf0, obuf1,
          gsem0, gsem1, osem0, osem1):
    gbuf = (gbuf0, gbuf1)
    obuf = (obuf0, obuf1)
    gsem = (gsem0, gsem1)
    osem = (osem0, osem1)

    wid = lax.axis_index("s") * NUM_CORES + lax.axis_index("c")
    base_tok = wid * TOK_PER_W
    base_row = base_tok // STEP

    pltpu.sync_copy(tok_hbm.at[pl.ds(pl.multiple_of(base_row, 8), IDX_ROWS)],
                    idx_all)

    def fire_gather(h, e):
        pltpu.async_copy(table_hbm.at[idx_all.at[h]], gbuf[e], gsem[e])

    def wait_gather(e):
        pltpu.make_async_copy(
            table_hbm.at[pl.ds(0, STEP)], gbuf[e], gsem[e]).wait()

    def fire_writeout(h, e):
        tok0 = pl.multiple_of(base_tok + h * STEP, 8)
        pltpu.async_copy(obuf[e], out_hbm.at[pl.ds(tok0, STEP)], osem[e])

    def wait_writeout(e):
        pltpu.make_async_copy(
            obuf[e], out_hbm.at[pl.ds(0, STEP)], osem[e]).wait()

    def scale(e):
        def sbody(r, c):
            for tt in range(4):
                for cc in range(EMB // 16):
                    src = (r * 4 + tt, pl.ds(cc * 16, 16))
                    obuf[e][src] = gbuf[e][src] * SCALE
            return c

        lax.fori_loop(0, STEP // 4, sbody, 0, unroll=False)

    def pipe_step(h, e, first_round):
        wait_gather(e)
        fire_gather(h + 1, 1 - e)
        if not first_round:
            wait_writeout(e)
        scale(e)
        fire_writeout(h, e)

    fire_gather(0, 0)
    pipe_step(0, 0, True)
    pipe_step(1, 1, True)

    def loop_body(i, c):
        h = i * 2
        pipe_step(h, 0, False)
        pipe_step(h + 1, 1, False)
        return c

    lax.fori_loop(1, (STEPS - 2) // 2, loop_body, 0, unroll=False)

    h = STEPS - 2
    wait_gather(0)
    fire_gather(h + 1, 1)
    wait_writeout(0)
    scale(0)
    fire_writeout(h, 0)

    wait_gather(1)
    wait_writeout(1)
    scale(1)
    fire_writeout(h + 1, 1)

    wait_writeout(0)
    wait_writeout(1)


def kernel(tokens, table):
    b, s = tokens.shape
    n = b * s
    tpad = jnp.pad(table, ((0, 0), (0, PADW - EMB)))
    tok2d = tokens.astype(jnp.int32).reshape(n // STEP, STEP)

    mesh = plsc.VectorSubcoreMesh(
        core_axis_name="c", subcore_axis_name="s",
        num_cores=NUM_CORES, num_subcores=NUM_SUBCORES,
    )
    out = pl.kernel(
        _body,
        out_type=jax.ShapeDtypeStruct((n, EMB), jnp.float32),
        mesh=mesh,
        compiler_params=pltpu.CompilerParams(use_tc_tiling_on_sc=True),
        scratch_types=[
            pltpu.VMEM((IDX_ROWS, STEP), jnp.int32),
            pltpu.VMEM((STEP, PADW), jnp.float32),
            pltpu.VMEM((STEP, PADW), jnp.float32),
            pltpu.VMEM((STEP, EMB), jnp.float32),
            pltpu.VMEM((STEP, EMB), jnp.float32),
            pltpu.SemaphoreType.DMA,
            pltpu.SemaphoreType.DMA,
            pltpu.SemaphoreType.DMA,
            pltpu.SemaphoreType.DMA,
        ],
    )(tpad, tok2d)
    return out.reshape(b, s, EMB)
